# baseline (speedup 1.0000x reference)
import functools

import jax
import jax.numpy as jnp
from jax import lax
from jax.experimental import pallas as pl
from jax.experimental.pallas import tpu as pltpu
from jax.experimental.pallas import tpu_sc as plsc

NQ = 16
ND = 32
NCAND = 1000000
K = 100

BLK = 131072
NBLK = 8
NPAD = NBLK * BLK
CHUNK = 128
NCHUNK = NPAD // CHUNK
L2N = NCHUNK // 16
L3N = 32
P2_L2N = (K * CHUNK) // 16
P2_L2PAD = 1024
P2_L3N = 64

NEG = float("-inf")


def _score_body(q_ref, c_ref, scores_ref, cmax_ref):
    i = pl.program_id(0)
    q = q_ref[...]
    c = c_ref[...]
    s = lax.dot_general(q, c, (((1,), (0,)), ((), ())),
                        preferred_element_type=jnp.float32)
    gidx = i * BLK + lax.broadcasted_iota(jnp.int32, (NQ, BLK), 1)
    s = jnp.where(gidx < NCAND, s, NEG)
    s3 = s.reshape(NQ, BLK // CHUNK, CHUNK)
    scores_ref[...] = s3
    cmax_ref[...] = jnp.max(s3, axis=2)


def _stage_a(queries, candidates):
    return pl.pallas_call(
        _score_body,
        grid=(NBLK,),
        in_specs=[
            pl.BlockSpec((NQ, ND), lambda i: (0, 0)),
            pl.BlockSpec((ND, BLK), lambda i: (0, i)),
        ],
        out_specs=[
            pl.BlockSpec((NQ, BLK // CHUNK, CHUNK), lambda i: (0, i, 0)),
            pl.BlockSpec((NQ, BLK // CHUNK), lambda i: (0, i)),
        ],
        out_shape=[
            jax.ShapeDtypeStruct((NQ, NCHUNK, CHUNK), jnp.float32),
            jax.ShapeDtypeStruct((NQ, NCHUNK), jnp.float32),
        ],
        compiler_params=pltpu.CompilerParams(
            dimension_semantics=("arbitrary",),
            vmem_limit_bytes=61440 * 1024),
    )(queries, candidates)


def _ffsv(mask):
    return plsc.all_reduce_ffs(mask)


def _pcv(mask):
    return plsc.all_reduce_population_count(mask)


def _hmax(v):
    return lax.reduce_max(v, (0,))


def _load1(ref, i):
    return plsc.load_gather(ref, [jnp.full((16,), i, jnp.int32)])[0]


def _store1(ref, i, val, lane):
    idx = jnp.full((16,), i, jnp.int32)
    v = jnp.full((16,), val)
    plsc.store_scatter(ref, [idx], v, mask=lane == 0)


def _cnt_better(lst, x, use_gt):
    c = jnp.zeros((16,), jnp.int32)
    for s in (64, 32, 16, 8, 4, 2, 1):
        v = plsc.load_gather(lst, [c + (s - 1)])
        cond = jnp.where(use_gt, v > x, v >= x)
        c = c + jnp.where(cond, s, 0)
    return c


HALF = NCHUNK // 2
P2C = 50
P2N = P2C * CHUNK
P2H_L2N = P2N // 16


def _sel_body(scores_hbm, cmax_hbm, ids_hbm, vals_out, idx_out,
              cm_buf, l2_buf, chunk_ids, cval_buf, pids_buf, pval_buf,
              p2cids, score_buf, p2l2, val_buf, idx_buf,
              mval_buf, midx_buf, gath_ids, shv, shi, sem):
    cid = lax.axis_index("c")
    sid = lax.axis_index("s")
    q = cid * 8 + (sid >> 1)
    half = sid & 1
    is_a = half == 0
    lane = lax.iota(jnp.int32, 16)
    neg16 = jnp.full((16,), NEG, jnp.float32)

    pltpu.sync_copy(cmax_hbm.at[q, pl.ds(half * HALF, HALF)], cm_buf)
    for g in range(8):
        chunk_ids[pl.ds(g * 16, 16)] = jnp.zeros((16,), jnp.int32)
        cval_buf[pl.ds(g * 16, 16)] = neg16
        idx_buf[pl.ds(g * 16, 16)] = jnp.zeros((16,), jnp.int32)
        val_buf[pl.ds(g * 16, 16)] = neg16
        midx_buf[pl.ds(g * 16, 16)] = jnp.zeros((16,), jnp.int32)
        mval_buf[pl.ds(g * 16, 16)] = jnp.zeros((16,), jnp.float32)
    for g in range(4):
        p2cids[pl.ds(g * 16, 16)] = jnp.zeros((16,), jnp.int32)

    def build_l2(g, _):
        acc = neg16
        for j in range(16):
            v = plsc.load_gather(cm_buf, [lane * 16 + g * 256 + j])
            acc = jnp.maximum(acc, v)
        l2_buf[pl.ds(g * 16, 16)] = acc
        return 0

    lax.fori_loop(0, HALF // 256, build_l2, 0)

    v0 = neg16
    for j in range(16):
        v0 = jnp.maximum(v0, plsc.load_gather(l2_buf, [lane * 16 + j]))

    def extract_chunk(t, v0):
        m = _hmax(v0)
        jv = _ffsv(v0 == m)
        u = plsc.load_gather(l2_buf, [jv * 16 + lane])
        iov = _ffsv(u == m)
        iv = jv * 16 + iov
        w = plsc.load_gather(cm_buf, [iv * 16 + lane])
        cov = _ffsv(w == m)
        tsplat = jnp.full((16,), t, jnp.int32)
        plsc.store_scatter(chunk_ids, [tsplat],
                           half * HALF + iv * 16 + cov, mask=lane == 0)
        plsc.store_scatter(cval_buf, [tsplat],
                           jnp.full((16,), m, jnp.float32), mask=lane == 0)
        w2 = jnp.where(lane == cov, NEG, w)
        plsc.store_scatter(cm_buf, [iv * 16 + lane], w2)
        um = jnp.where(lane == iov, NEG, u)
        nvw = _hmax(w2)
        nvuv = jnp.maximum(jnp.full((16,), _hmax(um), jnp.float32),
                           jnp.full((16,), nvw, jnp.float32))
        plsc.store_scatter(l2_buf, [jv * 16 + lane],
                           jnp.where(lane == iov, nvw, u))
        return jnp.where(lane == jv, nvuv, v0)

    lax.fori_loop(0, K, extract_chunk, v0)

    pltpu.sync_copy(cval_buf, shv.at[sid])
    pltpu.sync_copy(chunk_ids, shi.at[sid])
    plsc.subcore_barrier()
    pltpu.sync_copy(shv.at[sid ^ 1], pval_buf)
    pltpu.sync_copy(shi.at[sid ^ 1], pids_buf)
    plsc.subcore_barrier()

    lo = half * P2C
    for g in range(7):
        pos = lane + g * 16
        mine_v = plsc.load_gather(cval_buf, [pos])
        mine_i = plsc.load_gather(chunk_ids, [pos])
        r_own = pos + _cnt_better(pval_buf, mine_v, is_a)
        keep = jnp.logical_and(pos < K, jnp.logical_and(
            r_own >= lo, r_own < lo + P2C))
        plsc.store_scatter(p2cids, [r_own - lo], mine_i, mask=keep)
        part_v = plsc.load_gather(pval_buf, [pos])
        part_i = plsc.load_gather(pids_buf, [pos])
        r_par = pos + _cnt_better(cval_buf, part_v,
                                  jnp.logical_not(is_a))
        keepp = jnp.logical_and(pos < K, jnp.logical_and(
            r_par >= lo, r_par < lo + P2C))
        plsc.store_scatter(p2cids, [r_par - lo], part_i, mask=keepp)

    pltpu.async_copy(scores_hbm.at[q].at[p2cids], score_buf, sem).wait()

    for g in range(P2H_L2N, 512, 16):
        p2l2[pl.ds(g, 16)] = neg16

    def build_p2l2(g, _):
        acc = neg16
        for j in range(16):
            f = lane * 16 + g * 256 + j
            v = plsc.load_gather(score_buf, [f >> 7, f & 127])
            acc = jnp.maximum(acc, v)
        p2l2[pl.ds(g * 16, 16)] = acc
        return 0

    lax.fori_loop(0, P2H_L2N // 16, build_p2l2, 0)

    def _l3_group(buf, g):
        acc = neg16
        for j in range(16):
            acc = jnp.maximum(
                acc, plsc.load_gather(buf, [lane * 16 + g * 256 + j]))
        return acc

    t0 = _l3_group(p2l2, 0)
    t1 = _l3_group(p2l2, 1)

    def extract_elem(t, carry):
        t0, t1 = carry
        m = _hmax(jnp.maximum(t0, t1))
        use_hi = _pcv(t0 == m) == 0
        grp = jnp.where(use_hi, t1, t0)
        joff = _ffsv(grp == m)
        jv = jnp.where(use_hi, 16, 0) + joff
        u = plsc.load_gather(p2l2, [jv * 16 + lane])
        iov = _ffsv(u == m)
        ev = jv * 16 + iov
        rowv = ev >> 3
        colv = (ev & 7) * 16 + lane
        w = plsc.load_gather(score_buf, [rowv, colv])
        cov = _ffsv(w == m)
        fv = ev * 16 + cov
        tsplat = jnp.full((16,), t, jnp.int32)
        plsc.store_scatter(val_buf, [tsplat],
                           jnp.full((16,), m, jnp.float32), mask=lane == 0)
        cidv = plsc.load_gather(p2cids, [fv >> 7])
        plsc.store_scatter(idx_buf, [tsplat],
                           cidv * CHUNK + (fv & 127), mask=lane == 0)
        w2 = jnp.where(lane == cov, NEG, w)
        plsc.store_scatter(score_buf, [rowv, colv], w2)
        um = jnp.where(lane == iov, NEG, u)
        nvw = _hmax(w2)
        nvuv = jnp.maximum(jnp.full((16,), _hmax(um), jnp.float32),
                           jnp.full((16,), nvw, jnp.float32))
        plsc.store_scatter(p2l2, [jv * 16 + lane],
                           jnp.where(lane == iov, nvw, u))
        upd = lane == joff
        t0n = jnp.where(jnp.logical_and(jnp.logical_not(use_hi), upd),
                        nvuv, t0)
        t1n = jnp.where(jnp.logical_and(use_hi, upd), nvuv, t1)
        return (t0n, t1n)

    lax.fori_loop(0, K, extract_elem, (t0, t1))

    pltpu.sync_copy(val_buf, shv.at[sid])
    pltpu.sync_copy(idx_buf, shi.at[sid])
    plsc.subcore_barrier()
    pltpu.sync_copy(shv.at[sid ^ 1], pval_buf)
    pltpu.sync_copy(shi.at[sid ^ 1], pids_buf)

    @pl.when(is_a)
    def _():
        for g in range(7):
            pos = lane + g * 16
            mine_v = plsc.load_gather(val_buf, [pos])
            mine_i = plsc.load_gather(idx_buf, [pos])
            r_own = pos + _cnt_better(pval_buf, mine_v, True)
            keep = jnp.logical_and(pos < K, r_own < K)
            plsc.store_scatter(mval_buf, [r_own], mine_v, mask=keep)
            plsc.store_scatter(midx_buf, [r_own], mine_i, mask=keep)
            part_v = plsc.load_gather(pval_buf, [pos])
            part_i = plsc.load_gather(pids_buf, [pos])
            r_par = pos + _cnt_better(val_buf, part_v, False)
            keepp = jnp.logical_and(pos < K, r_par < K)
            plsc.store_scatter(mval_buf, [r_par], part_v, mask=keepp)
            plsc.store_scatter(midx_buf, [r_par], part_i, mask=keepp)

        pltpu.async_copy(ids_hbm.at[midx_buf], gath_ids, sem).wait()
        pltpu.sync_copy(mval_buf, vals_out.at[q])
        pltpu.sync_copy(gath_ids, idx_out.at[q])


def _stage_b(scores3, cmax, identifiers):
    mesh = plsc.VectorSubcoreMesh(core_axis_name="c", subcore_axis_name="s")
    kfn = pl.kernel(
        _sel_body,
        out_type=[
            jax.ShapeDtypeStruct((NQ, 128), jnp.float32),
            jax.ShapeDtypeStruct((NQ, 128), jnp.int32),
        ],
        mesh=mesh,
        scratch_types=[
            pltpu.VMEM((HALF,), jnp.float32),
            pltpu.VMEM((HALF // 16,), jnp.float32),
            pltpu.VMEM((128,), jnp.int32),
            pltpu.VMEM((128,), jnp.float32),
            pltpu.VMEM((128,), jnp.int32),
            pltpu.VMEM((128,), jnp.float32),
            pltpu.VMEM((64,), jnp.int32),
            pltpu.VMEM((64, CHUNK), jnp.float32),
            pltpu.VMEM((512,), jnp.float32),
            pltpu.VMEM((128,), jnp.float32),
            pltpu.VMEM((128,), jnp.int32),
            pltpu.VMEM((128,), jnp.float32),
            pltpu.VMEM((128,), jnp.int32),
            pltpu.VMEM((128,), jnp.int32),
            pltpu.VMEM_SHARED((16, 128), jnp.float32),
            pltpu.VMEM_SHARED((16, 128), jnp.int32),
            pltpu.SemaphoreType.DMA,
        ],
        compiler_params=pltpu.CompilerParams(needs_layout_passes=False),
    )
    return kfn(scores3, cmax, identifiers)


def kernel(queries, candidates, identifiers, k):
    scores3, cmax = _stage_a(queries, candidates.T)
    vals, idx = _stage_b(scores3, cmax, identifiers)
    return (vals[:, :K], idx[:, :K])


# speedup vs baseline: 1.0288x; 1.0288x over previous
import jax
import jax.numpy as jnp
from jax import lax
from jax.experimental import pallas as pl
from jax.experimental.pallas import tpu as pltpu
from jax.experimental.pallas import tpu_sc as plsc

NQ = 16
ND = 32
NCAND = 1000000
K = 100

BLK = 131072
NBLK = 8
NPAD = NBLK * BLK
CHUNK = 128
NCHUNK = NPAD // CHUNK
L2N = NCHUNK // 16
P2_L2N = (K * CHUNK) // 16
P2_L2PAD = 1024

NEG = float("-inf")


def _score_body(q_ref, c_ref, scores_ref, cmax_ref):
    i = pl.program_id(0)
    q = q_ref[...]
    c = c_ref[...]
    s = lax.dot_general(q, c, (((1,), (0,)), ((), ())),
                        preferred_element_type=jnp.float32)
    gidx = i * BLK + lax.broadcasted_iota(jnp.int32, (NQ, BLK), 1)
    s = jnp.where(gidx < NCAND, s, NEG)
    s3 = s.reshape(NQ, BLK // CHUNK, CHUNK)
    scores_ref[...] = s3
    cmax_ref[...] = jnp.max(s3, axis=2)


def _stage_a(queries, candidates):
    return pl.pallas_call(
        _score_body,
        grid=(NBLK,),
        in_specs=[
            pl.BlockSpec((NQ, ND), lambda i: (0, 0)),
            pl.BlockSpec((ND, BLK), lambda i: (0, i)),
        ],
        out_specs=[
            pl.BlockSpec((NQ, BLK // CHUNK, CHUNK), lambda i: (0, i, 0)),
            pl.BlockSpec((NQ, BLK // CHUNK), lambda i: (0, i)),
        ],
        out_shape=[
            jax.ShapeDtypeStruct((NQ, NCHUNK, CHUNK), jnp.float32),
            jax.ShapeDtypeStruct((NQ, NCHUNK), jnp.float32),
        ],
        compiler_params=pltpu.CompilerParams(
            dimension_semantics=("arbitrary",),
            vmem_limit_bytes=61440 * 1024),
    )(queries, candidates)


def _ffsv(mask):
    return plsc.all_reduce_ffs(mask)


def _pcv(mask):
    return plsc.all_reduce_population_count(mask)


def _hmax(v):
    return lax.reduce_max(v, (0,))


def _sel_body(scores_hbm, cmax_hbm, ids_hbm, vals_out, idx_out,
              cm_buf, l2_buf, chunk_ids, score_buf,
              p2l2, val_buf, idx_buf, gath_ids, sem):
    cid = lax.axis_index("c")
    sid = lax.axis_index("s")
    q = sid
    lane = lax.iota(jnp.int32, 16)
    neg16 = jnp.full((16,), NEG, jnp.float32)

    @pl.when(cid == 0)
    def _():
        pltpu.sync_copy(cmax_hbm.at[q], cm_buf)
        l2_buf[pl.ds(L2N, 16)] = neg16
        for g in range(8):
            chunk_ids[pl.ds(g * 16, 16)] = jnp.zeros((16,), jnp.int32)
            idx_buf[pl.ds(g * 16, 16)] = jnp.zeros((16,), jnp.int32)
            val_buf[pl.ds(g * 16, 16)] = jnp.zeros((16,), jnp.float32)

        def build_l2(g, _):
            acc = neg16
            for j in range(16):
                v = plsc.load_gather(cm_buf, [lane * 16 + g * 256 + j])
                acc = jnp.maximum(acc, v)
            l2_buf[pl.ds(g * 16, 16)] = acc
            return 0

        lax.fori_loop(0, L2N // 16, build_l2, 0)

        def _l3_group(buf, g):
            acc = neg16
            for j in range(16):
                acc = jnp.maximum(
                    acc, plsc.load_gather(buf, [lane * 16 + g * 256 + j]))
            return acc

        v0 = _l3_group(l2_buf, 0)
        v1 = _l3_group(l2_buf, 1)

        def extract_chunk(t, carry):
            v0, v1 = carry
            m = _hmax(jnp.maximum(v0, v1))
            use_hi = _pcv(v0 == m) == 0
            grp = jnp.where(use_hi, v1, v0)
            jv = jnp.where(use_hi, 16, 0) + _ffsv(grp == m)
            u = plsc.load_gather(l2_buf, [jv * 16 + lane])
            iov = _ffsv(u == m)
            iv = jv * 16 + iov
            w = plsc.load_gather(cm_buf, [iv * 16 + lane])
            cov = _ffsv(w == m)
            plsc.store_scatter(chunk_ids, [jnp.full((16,), t, jnp.int32)],
                               iv * 16 + cov, mask=lane == 0)
            w2 = jnp.where(lane == cov, NEG, w)
            plsc.store_scatter(cm_buf, [iv * 16 + lane], w2)
            um = jnp.where(lane == iov, NEG, u)
            nvw = _hmax(w2)
            nvuv = jnp.maximum(jnp.full((16,), _hmax(um), jnp.float32),
                               jnp.full((16,), nvw, jnp.float32))
            plsc.store_scatter(l2_buf, [jv * 16 + lane],
                               jnp.where(lane == iov, nvw, u))
            joff = _ffsv(grp == m)
            upd = lane == joff
            v0n = jnp.where(jnp.logical_and(jnp.logical_not(use_hi), upd),
                            nvuv, v0)
            v1n = jnp.where(jnp.logical_and(use_hi, upd), nvuv, v1)
            return (v0n, v1n)

        carry = lax.fori_loop(0, 64, extract_chunk, (v0, v1))
        cpA = pltpu.async_copy(scores_hbm.at[q].at[chunk_ids.at[pl.ds(0, 64)]],
                               score_buf.at[pl.ds(0, 64)], sem)
        lax.fori_loop(64, K, extract_chunk, carry)

        cpB = pltpu.async_copy(scores_hbm.at[q].at[chunk_ids.at[pl.ds(64, 64)]],
                               score_buf.at[pl.ds(64, 64)], sem)
        cpA.wait()
        cpB.wait()

        for g in range(P2_L2N, P2_L2PAD, 16):
            p2l2[pl.ds(g, 16)] = neg16

        def build_p2l2(g, _):
            acc = neg16
            for j in range(16):
                f = lane * 16 + g * 256 + j
                v = plsc.load_gather(score_buf, [f >> 7, f & 127])
                acc = jnp.maximum(acc, v)
            p2l2[pl.ds(g * 16, 16)] = acc
            return 0

        lax.fori_loop(0, P2_L2N // 16, build_p2l2, 0)

        t0 = _l3_group(p2l2, 0)
        t1 = _l3_group(p2l2, 1)
        t2 = _l3_group(p2l2, 2)
        t3 = _l3_group(p2l2, 3)

        def extract_elem(t, carry):
            t0, t1, t2, t3 = carry
            m = _hmax(jnp.maximum(jnp.maximum(t0, t1), jnp.maximum(t2, t3)))
            p0 = _pcv(t0 == m) > 0
            p1 = _pcv(t1 == m) > 0
            p2 = _pcv(t2 == m) > 0
            gv = jnp.where(p0, 0, jnp.where(p1, 1, jnp.where(p2, 2, 3)))
            grp = jnp.where(p0, t0, jnp.where(p1, t1, jnp.where(p2, t2, t3)))
            joff = _ffsv(grp == m)
            jv = gv * 16 + joff
            u = plsc.load_gather(p2l2, [jv * 16 + lane])
            iov = _ffsv(u == m)
            ev = jv * 16 + iov
            rowv = ev >> 3
            colv = (ev & 7) * 16 + lane
            w = plsc.load_gather(score_buf, [rowv, colv])
            cov = _ffsv(w == m)
            fv = ev * 16 + cov
            tsplat = jnp.full((16,), t, jnp.int32)
            plsc.store_scatter(val_buf, [tsplat],
                               jnp.full((16,), m, jnp.float32), mask=lane == 0)
            cidx = plsc.load_gather(chunk_ids, [fv >> 7])
            plsc.store_scatter(idx_buf, [tsplat],
                               cidx * CHUNK + (fv & 127), mask=lane == 0)
            w2 = jnp.where(lane == cov, NEG, w)
            plsc.store_scatter(score_buf, [rowv, colv], w2)
            um = jnp.where(lane == iov, NEG, u)
            nvw = _hmax(w2)
            nvuv = jnp.maximum(jnp.full((16,), _hmax(um), jnp.float32),
                               jnp.full((16,), nvw, jnp.float32))
            plsc.store_scatter(p2l2, [jv * 16 + lane],
                               jnp.where(lane == iov, nvw, u))
            upd = lane == joff
            np0 = jnp.logical_not(p0)
            np1 = jnp.logical_not(p1)
            t0n = jnp.where(jnp.logical_and(p0, upd), nvuv, t0)
            t1n = jnp.where(jnp.logical_and(jnp.logical_and(np0, p1), upd),
                            nvuv, t1)
            t2n = jnp.where(
                jnp.logical_and(jnp.logical_and(np0, jnp.logical_and(np1, p2)),
                                upd), nvuv, t2)
            t3n = jnp.where(
                jnp.logical_and(
                    jnp.logical_and(np0, jnp.logical_and(
                        np1, jnp.logical_not(p2))), upd), nvuv, t3)
            return (t0n, t1n, t2n, t3n)

        lax.fori_loop(0, K, extract_elem, (t0, t1, t2, t3))

        pltpu.async_copy(ids_hbm.at[idx_buf], gath_ids, sem).wait()
        pltpu.sync_copy(val_buf, vals_out.at[q])
        pltpu.sync_copy(gath_ids, idx_out.at[q])


def _stage_b(scores3, cmax, identifiers):
    mesh = plsc.VectorSubcoreMesh(core_axis_name="c", subcore_axis_name="s")
    kfn = pl.kernel(
        _sel_body,
        out_type=[
            jax.ShapeDtypeStruct((NQ, 128), jnp.float32),
            jax.ShapeDtypeStruct((NQ, 128), jnp.int32),
        ],
        mesh=mesh,
        scratch_types=[
            pltpu.VMEM((NCHUNK,), jnp.float32),
            pltpu.VMEM((L2N + 16,), jnp.float32),
            pltpu.VMEM((128,), jnp.int32),
            pltpu.VMEM((128, CHUNK), jnp.float32),
            pltpu.VMEM((P2_L2PAD,), jnp.float32),
            pltpu.VMEM((128,), jnp.float32),
            pltpu.VMEM((128,), jnp.int32),
            pltpu.VMEM((128,), jnp.int32),
            pltpu.SemaphoreType.DMA,
        ],
        compiler_params=pltpu.CompilerParams(needs_layout_passes=False),
    )
    return kfn(scores3, cmax, identifiers)


def kernel(queries, candidates, identifiers, k):
    scores3, cmax = _stage_a(queries, candidates.T)
    vals, idx = _stage_b(scores3, cmax, identifiers)
    return (vals[:, :K], idx[:, :K])
